# Initial kernel scaffold; baseline (speedup 1.0000x reference)
import jax
import jax.numpy as jnp
from jax.experimental import pallas as pl


def kernel(H, We0, be0, We1, be1, We2, be2, Wd0, bd0, Wd1, bd1, Wd2, bd2, sm_rows, sm_cols, sm_vals, sp_rows, sp_cols, sp_vals):
    raise NotImplementedError("write your pallas kernel here")


# speedup vs baseline: 21.1149x; 21.1149x over previous
import jax
import jax.numpy as jnp
from jax.experimental import pallas as pl
from jax.experimental.pallas import tpu as pltpu

N = 255
NNZ_PAD = 768


def _band_coefs(rows_ref, cols_ref, vals_ref, p_mod, blk):
    rows = rows_ref[0:1, :]
    cols = cols_ref[0:1, :]
    vals = vals_ref[0:1, :]
    off = cols - rows
    onehot = (p_mod == rows).astype(jnp.float32)
    bands = [jnp.sum(onehot * (vals * (off == o).astype(jnp.float32)),
                     axis=1, keepdims=True) for o in (-1, 0, 1)]
    return jnp.concatenate(bands + [jnp.zeros((blk, 5), jnp.float32)], axis=1)


def _band_apply(coef, y):
    return (coef[:, 0:1] * jnp.roll(y, 1, axis=0)
            + coef[:, 1:2] * y
            + coef[:, 2:3] * jnp.roll(y, -1, axis=0))


def _make_body(bc):
    blk = bc * N

    def body(sm_r, sm_c, sm_v, sp_r, sp_c, sp_v, h_ref,
             We0, be0, We1, be1, We2, be2, Wd0, bd0, Wd1, bd1, Wd2, bd2,
             out_ref, coef_sm, coef_sp):
        @pl.when(pl.program_id(0) == 0)
        def _():
            p_mod = jax.lax.broadcasted_iota(jnp.int32, (blk, 1), 0) % N
            coef_sm[...] = _band_coefs(sm_r, sm_c, sm_v, p_mod, blk)
            coef_sp[...] = _band_coefs(sp_r, sp_c, sp_v, p_mod, blk)

        csm = coef_sm[...]
        csp = coef_sp[...]
        x = h_ref[...]
        for W, b, coef in ((We0, be0, csm), (We1, be1, csm), (We2, be2, csm),
                           (Wd0, bd0, csp), (Wd1, bd1, csp), (Wd2, bd2, csp)):
            y = jnp.dot(x, W[...], preferred_element_type=jnp.float32) + b[...]
            x = jnp.maximum(_band_apply(coef, y), 0.0)
        out_ref[...] = x

    return body


def kernel(H, We0, be0, We1, be1, We2, be2, Wd0, bd0, Wd1, bd1, Wd2, bd2,
           sm_rows, sm_cols, sm_vals, sp_rows, sp_cols, sp_vals):
    B = H.shape[0]
    bc = 8
    blk = bc * N

    def pad_nnz(a):
        return jnp.pad(a, (0, NNZ_PAD - a.shape[0])).reshape(1, NNZ_PAD)

    coo = [pad_nnz(a) for a in
           (sm_rows, sm_cols, sm_vals, sp_rows, sp_cols, sp_vals)]
    weights = [We0, We1, We2, Wd0, Wd1, Wd2]
    biases = [b.reshape(1, -1) for b in (be0, be1, be2, bd0, bd1, bd2)]

    full = lambda a: pl.BlockSpec(a.shape, lambda i: (0, 0))
    in_specs = ([full(a) for a in coo]
                + [pl.BlockSpec((blk, 2), lambda i: (i, 0))])
    for W, b in zip(weights, biases):
        in_specs += [full(W), full(b)]

    inputs = list(coo) + [H.reshape(B * N, 2)]
    for W, b in zip(weights, biases):
        inputs += [W, b]

    out = pl.pallas_call(
        _make_body(bc),
        grid=(B // bc,),
        in_specs=in_specs,
        out_specs=pl.BlockSpec((blk, 2), lambda i: (i, 0)),
        out_shape=jax.ShapeDtypeStruct((B * N, 2), jnp.float32),
        scratch_shapes=[pltpu.VMEM((blk, 8), jnp.float32),
                        pltpu.VMEM((blk, 8), jnp.float32)],
    )(*inputs)
    return out.reshape(B, N, 2)


# speedup vs baseline: 21.8077x; 1.0328x over previous
import jax
import jax.numpy as jnp
from jax.experimental import pallas as pl
from jax.experimental.pallas import tpu as pltpu

N = 255
NNZ_PAD = 768


def _band_coefs(rows_ref, cols_ref, vals_ref, bc):
    rows = rows_ref[0:1, :]
    cols = cols_ref[0:1, :]
    vals = vals_ref[0:1, :]
    off = cols - rows
    p = jax.lax.broadcasted_iota(jnp.int32, (N + 1, 1), 0)
    onehot = (p == rows).astype(jnp.float32)
    bands = [jnp.sum(onehot * (vals * (off == o).astype(jnp.float32)),
                     axis=1, keepdims=True) for o in (-1, 0, 1)]
    period = jnp.concatenate(
        bands + [jnp.zeros((N + 1, 5), jnp.float32)], axis=1)[:N]
    return jnp.concatenate([period] * bc, axis=0)


def _band_apply(coef, y):
    return (coef[:, 0:1] * jnp.roll(y, 1, axis=0)
            + coef[:, 1:2] * y
            + coef[:, 2:3] * jnp.roll(y, -1, axis=0))


def _make_body(bc):
    blk = bc * N

    def body(sm_r, sm_c, sm_v, sp_r, sp_c, sp_v, h_ref,
             We0, be0, We1, be1, We2, be2, Wd0, bd0, Wd1, bd1, Wd2, bd2,
             out_ref, coef_sm, coef_sp):
        @pl.when(pl.program_id(0) == 0)
        def _():
            coef_sm[...] = _band_coefs(sm_r, sm_c, sm_v, bc)
            coef_sp[...] = _band_coefs(sp_r, sp_c, sp_v, bc)

        csm = coef_sm[...]
        csp = coef_sp[...]
        x = h_ref[...]
        for W, b, coef in ((We0, be0, csm), (We1, be1, csm), (We2, be2, csm),
                           (Wd0, bd0, csp), (Wd1, bd1, csp), (Wd2, bd2, csp)):
            y = jnp.dot(x, W[...], preferred_element_type=jnp.float32) + b[...]
            x = jnp.maximum(_band_apply(coef, y), 0.0)
        out_ref[...] = x

    return body


def kernel(H, We0, be0, We1, be1, We2, be2, Wd0, bd0, Wd1, bd1, Wd2, bd2,
           sm_rows, sm_cols, sm_vals, sp_rows, sp_cols, sp_vals):
    B = H.shape[0]
    bc = 8
    blk = bc * N

    def pad_nnz(a):
        return jnp.pad(a, (0, NNZ_PAD - a.shape[0])).reshape(1, NNZ_PAD)

    coo = [pad_nnz(a) for a in
           (sm_rows, sm_cols, sm_vals, sp_rows, sp_cols, sp_vals)]
    weights = [We0, We1, We2, Wd0, Wd1, Wd2]
    biases = [b.reshape(1, -1) for b in (be0, be1, be2, bd0, bd1, bd2)]

    full = lambda a: pl.BlockSpec(a.shape, lambda i: (0, 0))
    in_specs = ([full(a) for a in coo]
                + [pl.BlockSpec((blk, 2), lambda i: (i, 0))])
    for W, b in zip(weights, biases):
        in_specs += [full(W), full(b)]

    inputs = list(coo) + [H.reshape(B * N, 2)]
    for W, b in zip(weights, biases):
        inputs += [W, b]

    out = pl.pallas_call(
        _make_body(bc),
        grid=(B // bc,),
        in_specs=in_specs,
        out_specs=pl.BlockSpec((blk, 2), lambda i: (i, 0)),
        out_shape=jax.ShapeDtypeStruct((B * N, 2), jnp.float32),
        scratch_shapes=[pltpu.VMEM((blk, 8), jnp.float32),
                        pltpu.VMEM((blk, 8), jnp.float32)],
    )(*inputs)
    return out.reshape(B, N, 2)


# speedup vs baseline: 22.2129x; 1.0186x over previous
import jax
import jax.numpy as jnp
from jax.experimental import pallas as pl
from jax.experimental.pallas import tpu as pltpu

N = 255
NNZ_PAD = 768


def _band_coefs(rows_ref, cols_ref, vals_ref, bc):
    rows = rows_ref[0:1, :]
    cols = cols_ref[0:1, :]
    vals = vals_ref[0:1, :]
    off = cols - rows
    p = jax.lax.broadcasted_iota(jnp.int32, (N + 1, 1), 0)
    onehot = (p == rows).astype(jnp.float32)
    bands = [jnp.sum(onehot * (vals * (off == o).astype(jnp.float32)),
                     axis=1, keepdims=True) for o in (-1, 0, 1)]
    period = jnp.concatenate(
        bands + [jnp.zeros((N + 1, 5), jnp.float32)], axis=1)[:N]
    return jnp.concatenate([period] * bc, axis=0)


def _band_apply(coef, y):
    return (coef[:, 0:1] * jnp.roll(y, 1, axis=0)
            + coef[:, 1:2] * y
            + coef[:, 2:3] * jnp.roll(y, -1, axis=0))


def _make_body(bc):
    blk = bc * N

    def body(sm_r, sm_c, sm_v, sp_r, sp_c, sp_v, h_ref,
             We0, be0, We1, be1, We2, be2, Wd0, bd0, Wd1, bd1, Wd2, bd2,
             out_ref, coef_sm, coef_sp):
        @pl.when(pl.program_id(0) == 0)
        def _():
            coef_sm[...] = _band_coefs(sm_r, sm_c, sm_v, bc)
            coef_sp[...] = _band_coefs(sp_r, sp_c, sp_v, bc)

        csm = coef_sm[...]
        csp = coef_sp[...]
        x = h_ref[...]
        for W, b, coef in ((We0, be0, csm), (We1, be1, csm), (We2, be2, csm),
                           (Wd0, bd0, csp), (Wd1, bd1, csp), (Wd2, bd2, csp)):
            y = jnp.dot(x, W[...], preferred_element_type=jnp.float32) + b[...]
            x = jnp.maximum(_band_apply(coef, y), 0.0)
        out_ref[...] = x

    return body


def kernel(H, We0, be0, We1, be1, We2, be2, Wd0, bd0, Wd1, bd1, Wd2, bd2,
           sm_rows, sm_cols, sm_vals, sp_rows, sp_cols, sp_vals):
    B = H.shape[0]
    bc = 16
    blk = bc * N

    def pad_nnz(a):
        return jnp.pad(a, (0, NNZ_PAD - a.shape[0])).reshape(1, NNZ_PAD)

    coo = [pad_nnz(a) for a in
           (sm_rows, sm_cols, sm_vals, sp_rows, sp_cols, sp_vals)]
    weights = [We0, We1, We2, Wd0, Wd1, Wd2]
    biases = [b.reshape(1, -1) for b in (be0, be1, be2, bd0, bd1, bd2)]

    full = lambda a: pl.BlockSpec(a.shape, lambda i: (0, 0))
    in_specs = ([full(a) for a in coo]
                + [pl.BlockSpec((blk, 2), lambda i: (i, 0))])
    for W, b in zip(weights, biases):
        in_specs += [full(W), full(b)]

    inputs = list(coo) + [H.reshape(B * N, 2)]
    for W, b in zip(weights, biases):
        inputs += [W, b]

    out = pl.pallas_call(
        _make_body(bc),
        grid=(B // bc,),
        in_specs=in_specs,
        out_specs=pl.BlockSpec((blk, 2), lambda i: (i, 0)),
        out_shape=jax.ShapeDtypeStruct((B * N, 2), jnp.float32),
        scratch_shapes=[pltpu.VMEM((blk, 8), jnp.float32),
                        pltpu.VMEM((blk, 8), jnp.float32)],
    )(*inputs)
    return out.reshape(B, N, 2)
